# baseline (speedup 1.0000x reference)
import functools

import jax
import jax.numpy as jnp
from jax import lax
from jax.experimental import pallas as pl
from jax.experimental.pallas import tpu as pltpu
from jax.experimental.pallas import tpu_sc as plsc

N = 2048
D = 1024
FF = 1024
E = 64
CAP = 40
ROWS = (E + 1) * CAP
ZROW = E * CAP
NW = 32
TPB = N // NW


def _router_body(x_ref, gw_ref, idx_ref, cnt_ref):
    x = x_ref[...]
    gw = gw_ref[...]
    logits = lax.dot_general(
        x, gw, (((1,), (1,)), ((), ())),
        preferred_element_type=jnp.float32)
    m = jnp.max(logits, axis=1, keepdims=True)
    cols = lax.broadcasted_iota(jnp.int32, (N, E), 1)
    eid = jnp.min(jnp.where(logits >= m, cols, E), axis=1, keepdims=True)
    oh = (cols == eid).astype(jnp.int32)
    csum = oh
    k = 1
    while k < N:
        shifted = jnp.concatenate(
            [jnp.zeros((k, E), jnp.int32), csum[:N - k]], axis=0)
        csum = csum + shifted
        k *= 2
    pos = jnp.sum(csum * oh, axis=1, keepdims=True) - 1
    g = eid * CAP + pos
    idx_ref[...] = jnp.where(pos < CAP, g, ZROW)
    cnt_ref[...] = jnp.sum(oh, axis=0, keepdims=True)


def _router(xf, gate_w):
    idx2, cnt2 = pl.pallas_call(
        _router_body,
        out_shape=[
            jax.ShapeDtypeStruct((N, 1), jnp.int32),
            jax.ShapeDtypeStruct((1, E), jnp.int32),
        ],
    )(xf, gate_w)
    return idx2.reshape(N), cnt2.reshape(E)


@functools.cache
def _sc_kernels():
    mesh = plsc.VectorSubcoreMesh(core_axis_name="c", subcore_axis_name="s")
    nc = mesh.num_cores

    @functools.partial(
        pl.kernel, mesh=mesh,
        out_type=jax.ShapeDtypeStruct((ROWS,), jnp.int32),
        scratch_types=[
            pltpu.VMEM((TPB,), jnp.int32),
            pltpu.VMEM((TPB,), jnp.int32),
            pltpu.SemaphoreType.DMA,
        ],
    )
    def tos_scatter(idx_hbm, tokid_hbm, tos_hbm, idx_v, val_v, sem):
        wid = lax.axis_index("s") * nc + lax.axis_index("c")
        base = wid * TPB
        pltpu.sync_copy(idx_hbm.at[pl.ds(base, TPB)], idx_v)
        pltpu.sync_copy(tokid_hbm.at[pl.ds(base, TPB)], val_v)
        pltpu.async_copy(val_v, tos_hbm.at[idx_v], sem).wait()

    @functools.partial(
        pl.kernel, mesh=mesh,
        out_type=jax.ShapeDtypeStruct((N, D), jnp.float32),
        scratch_types=[
            pltpu.VMEM((TPB,), jnp.int32),
            pltpu.VMEM((TPB, D), jnp.float32),
            pltpu.SemaphoreType.DMA,
        ],
    )
    def combine(eo_hbm, idx_hbm, y_hbm, idx_v, rows_v, sem):
        wid = lax.axis_index("s") * nc + lax.axis_index("c")
        base = wid * TPB
        pltpu.sync_copy(idx_hbm.at[pl.ds(base, TPB)], idx_v)
        pltpu.async_copy(eo_hbm.at[idx_v], rows_v, sem).wait()
        pltpu.sync_copy(rows_v, y_hbm.at[pl.ds(base, TPB)])

    return tos_scatter, combine


def _ffn_body(tos_ref, xf_ref, wg_ref, wu_ref, wo_ref, out_ref,
              x_buf, sem_in):
    e = pl.program_id(0)
    par = lax.rem(e, 2)
    nxt = lax.rem(e + 1, 2)

    def issue_gathers(step, buf):
        for c in range(CAP):
            tok = jnp.clip(tos_ref[step * CAP + c], 0, N - 1)
            pltpu.make_async_copy(
                xf_ref.at[pl.ds(tok, 1)],
                x_buf.at[buf, pl.ds(c, 1)],
                sem_in.at[buf],
            ).start()

    @pl.when(e == 0)
    def _prologue():
        issue_gathers(0, 0)

    @pl.when(e + 1 < E)
    def _prefetch():
        issue_gathers(e + 1, nxt)

    @pl.when(e < E)
    def _compute():
        pltpu.make_async_copy(
            xf_ref.at[pl.ds(0, CAP)],
            x_buf.at[par],
            sem_in.at[par],
        ).wait()
        xb = x_buf[par]
        wg = wg_ref[0]
        wu = wu_ref[0]
        wo = wo_ref[0]
        hg = lax.dot_general(xb, wg, (((1,), (1,)), ((), ())),
                             preferred_element_type=jnp.float32)
        hu = lax.dot_general(xb, wu, (((1,), (1,)), ((), ())),
                             preferred_element_type=jnp.float32)
        h = hg * jax.nn.sigmoid(hg) * hu
        out_ref[...] = lax.dot_general(h, wo, (((1,), (1,)), ((), ())),
                                       preferred_element_type=jnp.float32)

    @pl.when(e == E)
    def _zero():
        out_ref[...] = jnp.zeros((CAP, D), jnp.float32)


def _ffn(tos, xf, wi_gate, wi_up, wo, interpret=False):
    return pl.pallas_call(
        _ffn_body,
        grid=(E + 1,),
        in_specs=[
            pl.BlockSpec(memory_space=pltpu.MemorySpace.SMEM),
            pl.BlockSpec(memory_space=pltpu.MemorySpace.HBM),
            pl.BlockSpec((1, FF, D), lambda e: (jnp.minimum(e, E - 1), 0, 0)),
            pl.BlockSpec((1, FF, D), lambda e: (jnp.minimum(e, E - 1), 0, 0)),
            pl.BlockSpec((1, D, FF), lambda e: (jnp.minimum(e, E - 1), 0, 0)),
        ],
        out_specs=pl.BlockSpec((CAP, D), lambda e: (e, 0)),
        out_shape=jax.ShapeDtypeStruct((ROWS, D), jnp.float32),
        scratch_shapes=[
            pltpu.VMEM((2, CAP, D), jnp.float32),
            pltpu.SemaphoreType.DMA((2,)),
        ],
        interpret=interpret,
    )(tos, xf, wi_gate, wi_up, wo)


def kernel(x, gate_w, wi_gate, wi_up, wo):
    B, S, D_ = x.shape
    xf = x.reshape(N, D)
    tos_scatter, combine = _sc_kernels()
    idx, counts = _router(xf, gate_w)
    tokid = lax.iota(jnp.int32, N)
    tos = tos_scatter(idx, tokid)
    eo = _ffn(tos, xf, wi_gate, wi_up, wo)
    y = combine(eo, idx)
    return y.reshape(B, S, D_)


# speedup vs baseline: 1.0744x; 1.0744x over previous
import functools

import jax
import jax.numpy as jnp
from jax import lax
from jax.experimental import pallas as pl
from jax.experimental.pallas import tpu as pltpu
from jax.experimental.pallas import tpu_sc as plsc

N = 2048
D = 1024
FF = 1024
E = 64
CAP = 40
ROWS = (E + 1) * CAP
ZROW = E * CAP
NW = 32
TPB = N // NW
NCHUNK = 2
CHW = TPB // NCHUNK


def _router_body(x_ref, gw_ref, idx_ref):
    x = x_ref[...]
    gw = gw_ref[...]
    logits = lax.dot_general(
        x, gw, (((1,), (1,)), ((), ())),
        preferred_element_type=jnp.float32)
    m = jnp.max(logits, axis=1, keepdims=True)
    cols = lax.broadcasted_iota(jnp.int32, (N, E), 1)
    eid = jnp.min(jnp.where(logits >= m, cols, E), axis=1, keepdims=True)
    oh = (cols == eid).astype(jnp.int32)
    csum = oh
    k = 1
    while k < N:
        shifted = jnp.concatenate(
            [jnp.zeros((k, E), jnp.int32), csum[:N - k]], axis=0)
        csum = csum + shifted
        k *= 2
    pos = jnp.sum(csum * oh, axis=1, keepdims=True) - 1
    g = eid * CAP + pos
    idx_ref[...] = jnp.where(pos < CAP, g, ZROW)


def _router(xf, gate_w):
    idx2 = pl.pallas_call(
        _router_body,
        out_shape=jax.ShapeDtypeStruct((N, 1), jnp.int32),
    )(xf, gate_w)
    return idx2.reshape(N)


@functools.cache
def _sc_kernels():
    mesh = plsc.VectorSubcoreMesh(core_axis_name="c", subcore_axis_name="s")
    nc = mesh.num_cores

    @functools.partial(
        pl.kernel, mesh=mesh,
        out_type=jax.ShapeDtypeStruct((ROWS, D), jnp.float32),
        scratch_types=[
            pltpu.VMEM((CHW,), jnp.int32),
            pltpu.VMEM((CHW,), jnp.int32),
            pltpu.VMEM((TPB, D), jnp.float32),
            pltpu.SemaphoreType.DMA,
            pltpu.SemaphoreType.DMA,
        ],
    )
    def dispatch(xf_hbm, idx_hbm, out_hbm, idx_v0, idx_v1, rows_v, lsem, ssem):
        idx_vs = (idx_v0, idx_v1)
        wid = lax.axis_index("s") * nc + lax.axis_index("c")
        base = wid * TPB
        ld = []
        for k in range(NCHUNK):
            ld.append(pltpu.make_async_copy(
                xf_hbm.at[pl.ds(base + k * CHW, CHW)],
                rows_v.at[pl.ds(k * CHW, CHW)], lsem))
            ld[k].start()
            pltpu.sync_copy(idx_hbm.at[pl.ds(base + k * CHW, CHW)], idx_vs[k])
        st = []
        for k in range(NCHUNK):
            ld[k].wait()
            st.append(pltpu.make_async_copy(
                rows_v.at[pl.ds(k * CHW, CHW)],
                out_hbm.at[idx_vs[k]], ssem))
            st[k].start()
        for k in range(NCHUNK):
            st[k].wait()

    @functools.partial(
        pl.kernel, mesh=mesh,
        out_type=jax.ShapeDtypeStruct((N, D), jnp.float32),
        scratch_types=[
            pltpu.VMEM((CHW,), jnp.int32),
            pltpu.VMEM((CHW,), jnp.int32),
            pltpu.VMEM((TPB, D), jnp.float32),
            pltpu.SemaphoreType.DMA,
            pltpu.SemaphoreType.DMA,
        ],
    )
    def combine(eo_hbm, idx_hbm, y_hbm, idx_v0, idx_v1, rows_v, gsem, wsem):
        idx_vs = (idx_v0, idx_v1)
        wid = lax.axis_index("s") * nc + lax.axis_index("c")
        base = wid * TPB
        g = []
        for k in range(NCHUNK):
            pltpu.sync_copy(idx_hbm.at[pl.ds(base + k * CHW, CHW)], idx_vs[k])
            g.append(pltpu.make_async_copy(
                eo_hbm.at[idx_vs[k]],
                rows_v.at[pl.ds(k * CHW, CHW)], gsem))
            g[k].start()
        wr = []
        for k in range(NCHUNK):
            g[k].wait()
            wr.append(pltpu.make_async_copy(
                rows_v.at[pl.ds(k * CHW, CHW)],
                y_hbm.at[pl.ds(base + k * CHW, CHW)], wsem))
            wr[k].start()
        for k in range(NCHUNK):
            wr[k].wait()

    return dispatch, combine


def _ffn_body(disp_ref, wg_ref, wu_ref, wo_ref, out_ref):
    e = pl.program_id(0)

    @pl.when(e < E)
    def _compute():
        xb = disp_ref[...]
        wg = wg_ref[0]
        wu = wu_ref[0]
        wo = wo_ref[0]
        hg = lax.dot_general(xb, wg, (((1,), (1,)), ((), ())),
                             preferred_element_type=jnp.float32)
        hu = lax.dot_general(xb, wu, (((1,), (1,)), ((), ())),
                             preferred_element_type=jnp.float32)
        h = hg * jax.nn.sigmoid(hg) * hu
        out_ref[...] = lax.dot_general(h, wo, (((1,), (1,)), ((), ())),
                                       preferred_element_type=jnp.float32)

    @pl.when(e == E)
    def _zero():
        out_ref[...] = jnp.zeros((CAP, D), jnp.float32)


def _ffn(dispatched, wi_gate, wi_up, wo, interpret=False):
    return pl.pallas_call(
        _ffn_body,
        grid=(E + 1,),
        in_specs=[
            pl.BlockSpec((CAP, D), lambda e: (e, 0)),
            pl.BlockSpec((1, FF, D), lambda e: (jnp.minimum(e, E - 1), 0, 0)),
            pl.BlockSpec((1, FF, D), lambda e: (jnp.minimum(e, E - 1), 0, 0)),
            pl.BlockSpec((1, D, FF), lambda e: (jnp.minimum(e, E - 1), 0, 0)),
        ],
        out_specs=pl.BlockSpec((CAP, D), lambda e: (e, 0)),
        out_shape=jax.ShapeDtypeStruct((ROWS, D), jnp.float32),
        interpret=interpret,
    )(dispatched, wi_gate, wi_up, wo)


def kernel(x, gate_w, wi_gate, wi_up, wo):
    B, S, D_ = x.shape
    xf = x.reshape(N, D)
    dispatch, combine = _sc_kernels()
    idx = _router(xf, gate_w)
    dispatched = dispatch(xf, idx)
    eo = _ffn(dispatched, wi_gate, wi_up, wo)
    y = combine(eo, idx)
    return y.reshape(B, S, D_)


# speedup vs baseline: 1.0750x; 1.0005x over previous
import functools

import jax
import jax.numpy as jnp
from jax import lax
from jax.experimental import pallas as pl
from jax.experimental.pallas import tpu as pltpu
from jax.experimental.pallas import tpu_sc as plsc

N = 2048
D = 1024
FF = 1024
E = 64
CAP = 40
ROWS = (E + 1) * CAP
ZROW = E * CAP
NW = 32
TPB = N // NW


def _router_body(x_ref, gw_ref, idx_ref):
    x = x_ref[...]
    gw = gw_ref[...]
    logits = lax.dot_general(
        x, gw, (((1,), (1,)), ((), ())),
        preferred_element_type=jnp.float32)
    m = jnp.max(logits, axis=1, keepdims=True)
    cols = lax.broadcasted_iota(jnp.int32, (N, E), 1)
    eid = jnp.min(jnp.where(logits >= m, cols, E), axis=1, keepdims=True)
    oh = (cols == eid).astype(jnp.int32)
    csum = oh
    k = 1
    while k < N:
        shifted = jnp.concatenate(
            [jnp.zeros((k, E), jnp.int32), csum[:N - k]], axis=0)
        csum = csum + shifted
        k *= 2
    pos = jnp.sum(csum * oh, axis=1, keepdims=True) - 1
    g = eid * CAP + pos
    idx_ref[...] = jnp.where(pos < CAP, g, ZROW)


def _router(xf, gate_w):
    idx2 = pl.pallas_call(
        _router_body,
        out_shape=jax.ShapeDtypeStruct((N, 1), jnp.int32),
    )(xf, gate_w)
    return idx2.reshape(N)


@functools.cache
def _sc_kernels():
    mesh = plsc.VectorSubcoreMesh(core_axis_name="c", subcore_axis_name="s")
    nc = mesh.num_cores

    @functools.partial(
        pl.kernel, mesh=mesh,
        out_type=jax.ShapeDtypeStruct((ROWS, D), jnp.float32),
        scratch_types=[
            pltpu.VMEM((TPB,), jnp.int32),
            pltpu.VMEM((TPB, D), jnp.float32),
            pltpu.SemaphoreType.DMA,
        ],
    )
    def dispatch(xf_hbm, idx_hbm, out_hbm, idx_v, rows_v, sem):
        wid = lax.axis_index("s") * nc + lax.axis_index("c")
        base = wid * TPB
        pltpu.sync_copy(idx_hbm.at[pl.ds(base, TPB)], idx_v)
        pltpu.sync_copy(xf_hbm.at[pl.ds(base, TPB)], rows_v)
        pltpu.async_copy(rows_v, out_hbm.at[idx_v], sem).wait()

    @functools.partial(
        pl.kernel, mesh=mesh,
        out_type=jax.ShapeDtypeStruct((N, D), jnp.float32),
        scratch_types=[
            pltpu.VMEM((TPB,), jnp.int32),
            pltpu.VMEM((TPB, D), jnp.float32),
            pltpu.SemaphoreType.DMA,
        ],
    )
    def combine(eo_hbm, idx_hbm, y_hbm, idx_v, rows_v, sem):
        wid = lax.axis_index("s") * nc + lax.axis_index("c")
        base = wid * TPB
        pltpu.sync_copy(idx_hbm.at[pl.ds(base, TPB)], idx_v)
        pltpu.async_copy(eo_hbm.at[idx_v], rows_v, sem).wait()
        pltpu.sync_copy(rows_v, y_hbm.at[pl.ds(base, TPB)])

    return dispatch, combine


def _ffn_body(disp_ref, wg_ref, wu_ref, wo_ref, out_ref):
    e = pl.program_id(0)

    @pl.when(e < E)
    def _compute():
        xb = disp_ref[...]
        wg = wg_ref[0]
        wu = wu_ref[0]
        wo = wo_ref[0]
        hg = lax.dot_general(xb, wg, (((1,), (1,)), ((), ())),
                             preferred_element_type=jnp.float32)
        hu = lax.dot_general(xb, wu, (((1,), (1,)), ((), ())),
                             preferred_element_type=jnp.float32)
        h = hg * jax.nn.sigmoid(hg) * hu
        out_ref[...] = lax.dot_general(h, wo, (((1,), (1,)), ((), ())),
                                       preferred_element_type=jnp.float32)

    @pl.when(e == E)
    def _zero():
        out_ref[...] = jnp.zeros((CAP, D), jnp.float32)


def _ffn(dispatched, wi_gate, wi_up, wo, interpret=False):
    return pl.pallas_call(
        _ffn_body,
        grid=(E + 1,),
        in_specs=[
            pl.BlockSpec((CAP, D), lambda e: (e, 0)),
            pl.BlockSpec((1, FF, D), lambda e: (jnp.minimum(e, E - 1), 0, 0)),
            pl.BlockSpec((1, FF, D), lambda e: (jnp.minimum(e, E - 1), 0, 0)),
            pl.BlockSpec((1, D, FF), lambda e: (jnp.minimum(e, E - 1), 0, 0)),
        ],
        out_specs=pl.BlockSpec((CAP, D), lambda e: (e, 0)),
        out_shape=jax.ShapeDtypeStruct((ROWS, D), jnp.float32),
        interpret=interpret,
    )(dispatched, wi_gate, wi_up, wo)


def kernel(x, gate_w, wi_gate, wi_up, wo):
    B, S, D_ = x.shape
    xf = x.reshape(N, D)
    dispatch, combine = _sc_kernels()
    idx = _router(xf, gate_w)
    dispatched = dispatch(xf, idx)
    eo = _ffn(dispatched, wi_gate, wi_up, wo)
    y = combine(eo, idx)
    return y.reshape(B, S, D_)
